# Initial kernel scaffold; baseline (speedup 1.0000x reference)
import jax
import jax.numpy as jnp
from jax.experimental import pallas as pl


def kernel(point_cloud, vis_mask, W1, b1, W2, b2, W3, b3):
    raise NotImplementedError("write your pallas kernel here")


# speedup vs baseline: 28.5526x; 28.5526x over previous
import functools

import jax
import jax.numpy as jnp
from jax import lax
from jax.experimental import pallas as pl

K = 20
TILE = 128
SWEEPS = 6


def _jacobi_rotate(A, V, p, q):
    apq = A[(p, q)]
    app = A[(p, p)]
    aqq = A[(q, q)]
    denom = 2.0 * apq
    safe = jnp.where(denom == 0.0, 1.0, denom)
    tau = jnp.where(denom == 0.0, 0.0, (aqq - app) / safe)
    t = jnp.sign(tau) / (jnp.abs(tau) + jnp.sqrt(1.0 + tau * tau))
    t = jnp.where(tau == 0.0, 1.0, t)
    t = jnp.where(apq == 0.0, 0.0, t)
    c = 1.0 / jnp.sqrt(1.0 + t * t)
    s = t * c
    r = 3 - p - q
    arp = A[(min(r, p), max(r, p))]
    arq = A[(min(r, q), max(r, q))]
    newA = dict(A)
    newA[(p, p)] = c * c * app - 2.0 * s * c * apq + s * s * aqq
    newA[(q, q)] = s * s * app + 2.0 * s * c * apq + c * c * aqq
    newA[(p, q)] = s * c * (app - aqq) + (c * c - s * s) * apq
    newA[(min(r, p), max(r, p))] = c * arp - s * arq
    newA[(min(r, q), max(r, q))] = s * arp + c * arq
    newV = dict(V)
    for i in range(3):
        vip = V[(i, p)]
        viq = V[(i, q)]
        newV[(i, p)] = c * vip - s * viq
        newV[(i, q)] = s * vip + c * viq
    return newA, newV


def _fused_kernel(ptsT_ref, pts_ref, sqr_ref, sqc_ref, W1_ref, b1_ref,
                  W2_ref, b2_ref, W3_ref, b3_ref, out_ref, *, n):
    ptsT = ptsT_ref[0]
    P = pts_ref[0]
    T = P.shape[0]
    sq_row = sqr_ref[0]
    sq_tile = sqc_ref[0]

    x_r = ptsT[0:1, :]
    y_r = ptsT[1:2, :]
    z_r = ptsT[2:3, :]

    Pb = P.astype(jnp.bfloat16)
    ptsTb = ptsT.astype(jnp.bfloat16)
    dot_tn = lax.dot_general(Pb, ptsTb, (((1,), (0,)), ((), ())),
                             preferred_element_type=jnp.float32)
    d2 = (sq_tile + sq_row) - 2.0 * dot_tn

    iota = lax.broadcasted_iota(jnp.int32, (T, n), 1)
    inf = jnp.float32(jnp.inf)
    d2w = d2
    wmask = jnp.zeros((T, n), jnp.bool_)
    dks = []
    for _ in range(K):
        m = jnp.min(d2w, axis=1, keepdims=True)
        am = jnp.min(jnp.where(d2w == m, iota, n), axis=1, keepdims=True)
        sel = iota == am
        wmask = jnp.logical_or(wmask, sel)
        d2w = jnp.where(sel, inf, d2w)
        dks.append(m)

    dists = [jnp.sqrt(jnp.maximum(dk, 0.0) + 1e-12) for dk in dks]
    dsum = dists[0]
    for dk in dists[1:]:
        dsum = dsum + dk
    mean_dist = dsum * (1.0 / K)
    max_dist = dists[-1]

    zero = jnp.float32(0.0)
    mx = jnp.sum(jnp.where(wmask, x_r, zero), axis=1, keepdims=True) / K
    my = jnp.sum(jnp.where(wmask, y_r, zero), axis=1, keepdims=True) / K
    mz = jnp.sum(jnp.where(wmask, z_r, zero), axis=1, keepdims=True) / K
    cx = (x_r - mx).astype(jnp.bfloat16).astype(jnp.float32)
    cy = (y_r - my).astype(jnp.bfloat16).astype(jnp.float32)
    cz = (z_r - mz).astype(jnp.bfloat16).astype(jnp.float32)

    def csum(a, b):
        return jnp.sum(jnp.where(wmask, a * b, zero), axis=1, keepdims=True) / K

    A = {
        (0, 0): csum(cx, cx),
        (1, 1): csum(cy, cy),
        (2, 2): csum(cz, cz),
        (0, 1): csum(cx, cy),
        (0, 2): csum(cx, cz),
        (1, 2): csum(cy, cz),
    }

    one = jnp.ones((T, 1), jnp.float32)
    zcol = jnp.zeros((T, 1), jnp.float32)
    V = {(i, j): (one if i == j else zcol) for i in range(3) for j in range(3)}
    for _ in range(SWEEPS):
        for (p, q) in ((0, 2), (1, 2), (0, 1)):
            A, V = _jacobi_rotate(A, V, p, q)

    cols = [
        (A[(0, 0)], V[(0, 0)], V[(1, 0)], V[(2, 0)]),
        (A[(1, 1)], V[(0, 1)], V[(1, 1)], V[(2, 1)]),
        (A[(2, 2)], V[(0, 2)], V[(1, 2)], V[(2, 2)]),
    ]

    def cswap(a, b):
        swap = a[0] > b[0]
        lo = tuple(jnp.where(swap, yv, xv) for xv, yv in zip(a, b))
        hi = tuple(jnp.where(swap, xv, yv) for xv, yv in zip(a, b))
        return lo, hi

    cols[0], cols[1] = cswap(cols[0], cols[1])
    cols[1], cols[2] = cswap(cols[1], cols[2])
    cols[0], cols[1] = cswap(cols[0], cols[1])

    lam0, nx, ny, nz = cols[0]
    lam1 = cols[1][0]
    lam2 = cols[2][0]
    curv = lam0 / ((lam0 + lam1 + lam2) + 1e-9)

    p0 = P[:, 0:1]
    p1 = P[:, 1:2]
    p2 = P[:, 2:3]
    x9 = jnp.concatenate(
        [p0, p1, p2, nx, ny, nz, curv, mean_dist, max_dist], axis=1)
    h1 = lax.dot_general(W1_ref[...], x9, (((1,), (1,)), ((), ())))
    h1 = jnp.maximum(h1 + b1_ref[...], 0.0)
    h2 = lax.dot_general(W2_ref[...], h1, (((1,), (0,)), ((), ())))
    h2 = jnp.maximum(h2 + b2_ref[...], 0.0)
    o = lax.dot_general(W3_ref[...], h2, (((1,), (0,)), ((), ())))
    out_ref[0] = o + b3_ref[...]


def kernel(point_cloud, vis_mask, W1, b1, W2, b2, W3, b3):
    B, N, _ = point_cloud.shape
    visible = jnp.where(vis_mask[..., None], point_cloud, jnp.zeros_like(point_cloud))
    ptsT = jnp.swapaxes(visible, 1, 2)
    sq = jnp.sum(visible * visible, axis=-1)
    sq_row = sq.reshape(B, 1, N)
    sq_col = sq.reshape(B, N, 1)
    C1, C2, C3 = W1.shape[0], W2.shape[0], W3.shape[0]
    b1c = b1.reshape(C1, 1)
    b2c = b2.reshape(C2, 1)
    b3c = b3.reshape(C3, 1)
    T = TILE
    grid = (B, N // T)
    out = pl.pallas_call(
        functools.partial(_fused_kernel, n=N),
        grid=grid,
        in_specs=[
            pl.BlockSpec((1, 3, N), lambda b, i: (b, 0, 0)),
            pl.BlockSpec((1, T, 3), lambda b, i: (b, i, 0)),
            pl.BlockSpec((1, 1, N), lambda b, i: (b, 0, 0)),
            pl.BlockSpec((1, T, 1), lambda b, i: (b, i, 0)),
            pl.BlockSpec((C1, 9), lambda b, i: (0, 0)),
            pl.BlockSpec((C1, 1), lambda b, i: (0, 0)),
            pl.BlockSpec((C2, C1), lambda b, i: (0, 0)),
            pl.BlockSpec((C2, 1), lambda b, i: (0, 0)),
            pl.BlockSpec((C3, C2), lambda b, i: (0, 0)),
            pl.BlockSpec((C3, 1), lambda b, i: (0, 0)),
        ],
        out_specs=pl.BlockSpec((1, C3, T), lambda b, i: (b, 0, i)),
        out_shape=jax.ShapeDtypeStruct((B, C3, N), jnp.float32),
    )(ptsT, visible, sq_row, sq_col, W1, b1c, W2, b2c, W3, b3c)
    return out


# speedup vs baseline: 45.9274x; 1.6085x over previous
import functools

import jax
import jax.numpy as jnp
from jax import lax
from jax.experimental import pallas as pl

K = 20
TILE = 128
SWEEPS = 6


def _jacobi_rotate(A, V, p, q):
    apq = A[(p, q)]
    app = A[(p, p)]
    aqq = A[(q, q)]
    denom = 2.0 * apq
    safe = jnp.where(denom == 0.0, 1.0, denom)
    tau = jnp.where(denom == 0.0, 0.0, (aqq - app) / safe)
    t = jnp.sign(tau) / (jnp.abs(tau) + jnp.sqrt(1.0 + tau * tau))
    t = jnp.where(tau == 0.0, 1.0, t)
    t = jnp.where(apq == 0.0, 0.0, t)
    c = 1.0 / jnp.sqrt(1.0 + t * t)
    s = t * c
    r = 3 - p - q
    arp = A[(min(r, p), max(r, p))]
    arq = A[(min(r, q), max(r, q))]
    newA = dict(A)
    newA[(p, p)] = c * c * app - 2.0 * s * c * apq + s * s * aqq
    newA[(q, q)] = s * s * app + 2.0 * s * c * apq + c * c * aqq
    newA[(p, q)] = s * c * (app - aqq) + (c * c - s * s) * apq
    newA[(min(r, p), max(r, p))] = c * arp - s * arq
    newA[(min(r, q), max(r, q))] = s * arp + c * arq
    newV = dict(V)
    for i in range(3):
        vip = V[(i, p)]
        viq = V[(i, q)]
        newV[(i, p)] = c * vip - s * viq
        newV[(i, q)] = s * vip + c * viq
    return newA, newV


def _fused_kernel(ptsT_ref, pts_ref, sqr_ref, sqc_ref, W1_ref, b1_ref,
                  W2_ref, b2_ref, W3_ref, b3_ref, out_ref, *, n):
    ptsT = ptsT_ref[0]
    P = pts_ref[0]
    T = P.shape[0]
    sq_row = sqr_ref[0]
    sq_tile = sqc_ref[0]

    x_r = ptsT[0:1, :]
    y_r = ptsT[1:2, :]
    z_r = ptsT[2:3, :]

    Pb = P.astype(jnp.bfloat16)
    ptsTb = ptsT.astype(jnp.bfloat16)
    dot_tn = lax.dot_general(Pb, ptsTb, (((1,), (0,)), ((), ())),
                             preferred_element_type=jnp.float32)
    d2 = (sq_tile + sq_row) - 2.0 * dot_tn

    iota = lax.broadcasted_iota(jnp.int32, (T, n), 1)
    inf = jnp.float32(jnp.inf)
    d2w = d2
    dks = []
    am = None
    for _ in range(K):
        m = jnp.min(d2w, axis=1, keepdims=True)
        am = jnp.min(jnp.where(d2w == m, iota, n), axis=1, keepdims=True)
        d2w = jnp.where(iota == am, inf, d2w)
        dks.append(m)

    v_cut = dks[-1]
    wmask = jnp.logical_or(d2 < v_cut,
                           jnp.logical_and(d2 == v_cut, iota <= am))

    zero = jnp.float32(0.0)
    mx = jnp.sum(jnp.where(wmask, x_r, zero), axis=1, keepdims=True) / K
    my = jnp.sum(jnp.where(wmask, y_r, zero), axis=1, keepdims=True) / K
    mz = jnp.sum(jnp.where(wmask, z_r, zero), axis=1, keepdims=True) / K
    cx = (x_r - mx).astype(jnp.bfloat16).astype(jnp.float32)
    cy = (y_r - my).astype(jnp.bfloat16).astype(jnp.float32)
    cz = (z_r - mz).astype(jnp.bfloat16).astype(jnp.float32)

    def csum(a, b):
        return jnp.sum(jnp.where(wmask, a * b, zero), axis=1, keepdims=True) / K

    p0 = P[:, 0:1]
    p1 = P[:, 1:2]
    p2 = P[:, 2:3]
    colcat = jnp.concatenate(
        [csum(cx, cx), csum(cy, cy), csum(cz, cz),
         csum(cx, cy), csum(cx, cz), csum(cy, cz),
         p0, p1, p2] + dks, axis=1)
    tr = jnp.transpose(colcat)
    A = {
        (0, 0): tr[0:1],
        (1, 1): tr[1:2],
        (2, 2): tr[2:3],
        (0, 1): tr[3:4],
        (0, 2): tr[4:5],
        (1, 2): tr[5:6],
    }
    p0r, p1r, p2r = tr[6:7], tr[7:8], tr[8:9]
    dmat = jnp.sqrt(jnp.maximum(tr[9:9 + K], 0.0) + 1e-12)
    mean_dist = jnp.sum(dmat, axis=0, keepdims=True) * (1.0 / K)
    max_dist = dmat[K - 1:K]

    one = jnp.ones((1, T), jnp.float32)
    zcol = jnp.zeros((1, T), jnp.float32)
    V = {(i, j): (one if i == j else zcol) for i in range(3) for j in range(3)}
    for _ in range(SWEEPS):
        for (p, q) in ((0, 2), (1, 2), (0, 1)):
            A, V = _jacobi_rotate(A, V, p, q)

    cols = [
        (A[(0, 0)], V[(0, 0)], V[(1, 0)], V[(2, 0)]),
        (A[(1, 1)], V[(0, 1)], V[(1, 1)], V[(2, 1)]),
        (A[(2, 2)], V[(0, 2)], V[(1, 2)], V[(2, 2)]),
    ]

    def cswap(a, b):
        swap = a[0] > b[0]
        lo = tuple(jnp.where(swap, yv, xv) for xv, yv in zip(a, b))
        hi = tuple(jnp.where(swap, xv, yv) for xv, yv in zip(a, b))
        return lo, hi

    cols[0], cols[1] = cswap(cols[0], cols[1])
    cols[1], cols[2] = cswap(cols[1], cols[2])
    cols[0], cols[1] = cswap(cols[0], cols[1])

    lam0, nx, ny, nz = cols[0]
    lam1 = cols[1][0]
    lam2 = cols[2][0]
    curv = lam0 / ((lam0 + lam1 + lam2) + 1e-9)

    xT = jnp.concatenate(
        [p0r, p1r, p2r, nx, ny, nz, curv, mean_dist, max_dist], axis=0)
    h1 = lax.dot_general(W1_ref[...], xT, (((1,), (0,)), ((), ())))
    h1 = jnp.maximum(h1 + b1_ref[...], 0.0)
    h2 = lax.dot_general(W2_ref[...], h1, (((1,), (0,)), ((), ())))
    h2 = jnp.maximum(h2 + b2_ref[...], 0.0)
    o = lax.dot_general(W3_ref[...], h2, (((1,), (0,)), ((), ())))
    out_ref[0] = o + b3_ref[...]


def kernel(point_cloud, vis_mask, W1, b1, W2, b2, W3, b3):
    B, N, _ = point_cloud.shape
    visible = jnp.where(vis_mask[..., None], point_cloud, jnp.zeros_like(point_cloud))
    ptsT = jnp.swapaxes(visible, 1, 2)
    sq = jnp.sum(visible * visible, axis=-1)
    sq_row = sq.reshape(B, 1, N)
    sq_col = sq.reshape(B, N, 1)
    C1, C2, C3 = W1.shape[0], W2.shape[0], W3.shape[0]
    b1c = b1.reshape(C1, 1)
    b2c = b2.reshape(C2, 1)
    b3c = b3.reshape(C3, 1)
    T = TILE
    grid = (B, N // T)
    out = pl.pallas_call(
        functools.partial(_fused_kernel, n=N),
        grid=grid,
        in_specs=[
            pl.BlockSpec((1, 3, N), lambda b, i: (b, 0, 0)),
            pl.BlockSpec((1, T, 3), lambda b, i: (b, i, 0)),
            pl.BlockSpec((1, 1, N), lambda b, i: (b, 0, 0)),
            pl.BlockSpec((1, T, 1), lambda b, i: (b, i, 0)),
            pl.BlockSpec((C1, 9), lambda b, i: (0, 0)),
            pl.BlockSpec((C1, 1), lambda b, i: (0, 0)),
            pl.BlockSpec((C2, C1), lambda b, i: (0, 0)),
            pl.BlockSpec((C2, 1), lambda b, i: (0, 0)),
            pl.BlockSpec((C3, C2), lambda b, i: (0, 0)),
            pl.BlockSpec((C3, 1), lambda b, i: (0, 0)),
        ],
        out_specs=pl.BlockSpec((1, C3, T), lambda b, i: (b, 0, i)),
        out_shape=jax.ShapeDtypeStruct((B, C3, N), jnp.float32),
    )(ptsT, visible, sq_row, sq_col, W1, b1c, W2, b2c, W3, b3c)
    return out


# speedup vs baseline: 60.5983x; 1.3194x over previous
import functools

import jax
import jax.numpy as jnp
from jax import lax
from jax.experimental import pallas as pl

K = 20
TILE = 256
SWEEPS = 6


def _jacobi_rotate(A, V, p, q):
    apq = A[(p, q)]
    app = A[(p, p)]
    aqq = A[(q, q)]
    denom = 2.0 * apq
    safe = jnp.where(denom == 0.0, 1.0, denom)
    tau = jnp.where(denom == 0.0, 0.0, (aqq - app) / safe)
    t = jnp.sign(tau) / (jnp.abs(tau) + jnp.sqrt(1.0 + tau * tau))
    t = jnp.where(tau == 0.0, 1.0, t)
    t = jnp.where(apq == 0.0, 0.0, t)
    c = 1.0 / jnp.sqrt(1.0 + t * t)
    s = t * c
    r = 3 - p - q
    arp = A[(min(r, p), max(r, p))]
    arq = A[(min(r, q), max(r, q))]
    newA = dict(A)
    newA[(p, p)] = c * c * app - 2.0 * s * c * apq + s * s * aqq
    newA[(q, q)] = s * s * app + 2.0 * s * c * apq + c * c * aqq
    newA[(p, q)] = s * c * (app - aqq) + (c * c - s * s) * apq
    newA[(min(r, p), max(r, p))] = c * arp - s * arq
    newA[(min(r, q), max(r, q))] = s * arp + c * arq
    newV = dict(V)
    for i in range(3):
        vip = V[(i, p)]
        viq = V[(i, q)]
        newV[(i, p)] = c * vip - s * viq
        newV[(i, q)] = s * vip + c * viq
    return newA, newV


def _fused_kernel(ptsT_ref, pts_ref, sqr_ref, sqc_ref, W1_ref, b1_ref,
                  W2_ref, b2_ref, W3_ref, b3_ref, out_ref, *, n):
    ptsT = ptsT_ref[0]
    P = pts_ref[0]
    T = P.shape[0]
    sq_row = sqr_ref[0]
    sq_tile = sqc_ref[0]

    x_r = ptsT[0:1, :]
    y_r = ptsT[1:2, :]
    z_r = ptsT[2:3, :]

    Pb = P.astype(jnp.bfloat16)
    ptsTb = ptsT.astype(jnp.bfloat16)
    dot_tn = lax.dot_general(Pb, ptsTb, (((1,), (0,)), ((), ())),
                             preferred_element_type=jnp.float32)
    d2 = (sq_tile + sq_row) - 2.0 * dot_tn

    iota = lax.broadcasted_iota(jnp.int32, (T, n), 1)
    inf = jnp.float32(jnp.inf)
    d2w = d2
    for _ in range(K - 1):
        am = jnp.argmin(d2w, axis=1, keepdims=True)
        d2w = jnp.where(iota == am, inf, d2w)
    v_cut = jnp.min(d2w, axis=1, keepdims=True)
    am = jnp.argmin(d2w, axis=1, keepdims=True)

    wmask = jnp.logical_or(d2 < v_cut,
                           jnp.logical_and(d2 == v_cut, iota <= am))

    zero = jnp.float32(0.0)
    mx = jnp.sum(jnp.where(wmask, x_r, zero), axis=1, keepdims=True) / K
    my = jnp.sum(jnp.where(wmask, y_r, zero), axis=1, keepdims=True) / K
    mz = jnp.sum(jnp.where(wmask, z_r, zero), axis=1, keepdims=True) / K
    cx = jnp.where(wmask, (x_r - mx).astype(jnp.bfloat16).astype(jnp.float32),
                   zero)
    cy = jnp.where(wmask, (y_r - my).astype(jnp.bfloat16).astype(jnp.float32),
                   zero)
    cz = jnp.where(wmask, (z_r - mz).astype(jnp.bfloat16).astype(jnp.float32),
                   zero)

    def csum(a, b):
        return jnp.sum(a * b, axis=1, keepdims=True) / K

    sqrtd = jnp.sqrt(jnp.maximum(d2, 0.0) + 1e-12)
    mean_d = jnp.sum(jnp.where(wmask, sqrtd, zero), axis=1, keepdims=True) / K
    max_d = jnp.sqrt(jnp.maximum(v_cut, 0.0) + 1e-12)

    p0 = P[:, 0:1]
    p1 = P[:, 1:2]
    p2 = P[:, 2:3]
    colcat = jnp.concatenate(
        [csum(cx, cx), csum(cy, cy), csum(cz, cz),
         csum(cx, cy), csum(cx, cz), csum(cy, cz),
         p0, p1, p2, mean_d, max_d], axis=1)
    tr = jnp.transpose(colcat)
    A = {
        (0, 0): tr[0:1],
        (1, 1): tr[1:2],
        (2, 2): tr[2:3],
        (0, 1): tr[3:4],
        (0, 2): tr[4:5],
        (1, 2): tr[5:6],
    }
    p0r, p1r, p2r = tr[6:7], tr[7:8], tr[8:9]
    mean_dist = tr[9:10]
    max_dist = tr[10:11]

    one = jnp.ones((1, T), jnp.float32)
    zcol = jnp.zeros((1, T), jnp.float32)
    V = {(i, j): (one if i == j else zcol) for i in range(3) for j in range(3)}
    for _ in range(SWEEPS):
        for (p, q) in ((0, 2), (1, 2), (0, 1)):
            A, V = _jacobi_rotate(A, V, p, q)

    cols = [
        (A[(0, 0)], V[(0, 0)], V[(1, 0)], V[(2, 0)]),
        (A[(1, 1)], V[(0, 1)], V[(1, 1)], V[(2, 1)]),
        (A[(2, 2)], V[(0, 2)], V[(1, 2)], V[(2, 2)]),
    ]

    def cswap(a, b):
        swap = a[0] > b[0]
        lo = tuple(jnp.where(swap, yv, xv) for xv, yv in zip(a, b))
        hi = tuple(jnp.where(swap, xv, yv) for xv, yv in zip(a, b))
        return lo, hi

    cols[0], cols[1] = cswap(cols[0], cols[1])
    cols[1], cols[2] = cswap(cols[1], cols[2])
    cols[0], cols[1] = cswap(cols[0], cols[1])

    lam0, nx, ny, nz = cols[0]
    lam1 = cols[1][0]
    lam2 = cols[2][0]
    curv = lam0 / ((lam0 + lam1 + lam2) + 1e-9)

    xT = jnp.concatenate(
        [p0r, p1r, p2r, nx, ny, nz, curv, mean_dist, max_dist], axis=0)
    h1 = lax.dot_general(W1_ref[...], xT, (((1,), (0,)), ((), ())))
    h1 = jnp.maximum(h1 + b1_ref[...], 0.0)
    h2 = lax.dot_general(W2_ref[...], h1, (((1,), (0,)), ((), ())))
    h2 = jnp.maximum(h2 + b2_ref[...], 0.0)
    o = lax.dot_general(W3_ref[...], h2, (((1,), (0,)), ((), ())))
    out_ref[0] = o + b3_ref[...]


def kernel(point_cloud, vis_mask, W1, b1, W2, b2, W3, b3):
    B, N, _ = point_cloud.shape
    visible = jnp.where(vis_mask[..., None], point_cloud, jnp.zeros_like(point_cloud))
    ptsT = jnp.swapaxes(visible, 1, 2)
    sq = jnp.sum(visible * visible, axis=-1)
    sq_row = sq.reshape(B, 1, N)
    sq_col = sq.reshape(B, N, 1)
    C1, C2, C3 = W1.shape[0], W2.shape[0], W3.shape[0]
    b1c = b1.reshape(C1, 1)
    b2c = b2.reshape(C2, 1)
    b3c = b3.reshape(C3, 1)
    T = TILE
    grid = (B, N // T)
    out = pl.pallas_call(
        functools.partial(_fused_kernel, n=N),
        grid=grid,
        in_specs=[
            pl.BlockSpec((1, 3, N), lambda b, i: (b, 0, 0)),
            pl.BlockSpec((1, T, 3), lambda b, i: (b, i, 0)),
            pl.BlockSpec((1, 1, N), lambda b, i: (b, 0, 0)),
            pl.BlockSpec((1, T, 1), lambda b, i: (b, i, 0)),
            pl.BlockSpec((C1, 9), lambda b, i: (0, 0)),
            pl.BlockSpec((C1, 1), lambda b, i: (0, 0)),
            pl.BlockSpec((C2, C1), lambda b, i: (0, 0)),
            pl.BlockSpec((C2, 1), lambda b, i: (0, 0)),
            pl.BlockSpec((C3, C2), lambda b, i: (0, 0)),
            pl.BlockSpec((C3, 1), lambda b, i: (0, 0)),
        ],
        out_specs=pl.BlockSpec((1, C3, T), lambda b, i: (b, 0, i)),
        out_shape=jax.ShapeDtypeStruct((B, C3, N), jnp.float32),
    )(ptsT, visible, sq_row, sq_col, W1, b1c, W2, b2c, W3, b3c)
    return out


# speedup vs baseline: 62.6553x; 1.0339x over previous
import functools

import jax
import jax.numpy as jnp
from jax import lax
from jax.experimental import pallas as pl

K = 20
TILE = 512
SWEEPS = 6


def _jacobi_rotate(A, V, p, q):
    apq = A[(p, q)]
    app = A[(p, p)]
    aqq = A[(q, q)]
    denom = 2.0 * apq
    safe = jnp.where(denom == 0.0, 1.0, denom)
    tau = jnp.where(denom == 0.0, 0.0, (aqq - app) / safe)
    t = jnp.sign(tau) / (jnp.abs(tau) + jnp.sqrt(1.0 + tau * tau))
    t = jnp.where(tau == 0.0, 1.0, t)
    t = jnp.where(apq == 0.0, 0.0, t)
    c = 1.0 / jnp.sqrt(1.0 + t * t)
    s = t * c
    r = 3 - p - q
    arp = A[(min(r, p), max(r, p))]
    arq = A[(min(r, q), max(r, q))]
    newA = dict(A)
    newA[(p, p)] = c * c * app - 2.0 * s * c * apq + s * s * aqq
    newA[(q, q)] = s * s * app + 2.0 * s * c * apq + c * c * aqq
    newA[(p, q)] = s * c * (app - aqq) + (c * c - s * s) * apq
    newA[(min(r, p), max(r, p))] = c * arp - s * arq
    newA[(min(r, q), max(r, q))] = s * arp + c * arq
    newV = dict(V)
    for i in range(3):
        vip = V[(i, p)]
        viq = V[(i, q)]
        newV[(i, p)] = c * vip - s * viq
        newV[(i, q)] = s * vip + c * viq
    return newA, newV


def _fused_kernel(ptsT_ref, pts_ref, sqr_ref, sqc_ref, W1_ref, b1_ref,
                  W2_ref, b2_ref, W3_ref, b3_ref, out_ref, *, n):
    ptsT = ptsT_ref[0]
    P = pts_ref[0]
    T = P.shape[0]
    sq_row = sqr_ref[0]
    sq_tile = sqc_ref[0]

    x_r = ptsT[0:1, :]
    y_r = ptsT[1:2, :]
    z_r = ptsT[2:3, :]

    Pb = P.astype(jnp.bfloat16)
    ptsTb = ptsT.astype(jnp.bfloat16)
    dot_tn = lax.dot_general(Pb, ptsTb, (((1,), (0,)), ((), ())),
                             preferred_element_type=jnp.float32)
    d2 = (sq_tile + sq_row) - 2.0 * dot_tn

    iota = lax.broadcasted_iota(jnp.int32, (T, n), 1)
    inf = jnp.float32(jnp.inf)
    d2w = d2
    for _ in range(K - 1):
        am = jnp.argmin(d2w, axis=1, keepdims=True)
        d2w = jnp.where(iota == am, inf, d2w)
    v_cut = jnp.min(d2w, axis=1, keepdims=True)
    am = jnp.argmin(d2w, axis=1, keepdims=True)

    wmask = jnp.logical_or(d2 < v_cut,
                           jnp.logical_and(d2 == v_cut, iota <= am))

    zero = jnp.float32(0.0)
    mx = jnp.sum(jnp.where(wmask, x_r, zero), axis=1, keepdims=True) / K
    my = jnp.sum(jnp.where(wmask, y_r, zero), axis=1, keepdims=True) / K
    mz = jnp.sum(jnp.where(wmask, z_r, zero), axis=1, keepdims=True) / K
    cx = jnp.where(wmask, (x_r - mx).astype(jnp.bfloat16).astype(jnp.float32),
                   zero)
    cy = jnp.where(wmask, (y_r - my).astype(jnp.bfloat16).astype(jnp.float32),
                   zero)
    cz = jnp.where(wmask, (z_r - mz).astype(jnp.bfloat16).astype(jnp.float32),
                   zero)

    def csum(a, b):
        return jnp.sum(a * b, axis=1, keepdims=True) / K

    sqrtd = jnp.sqrt(jnp.maximum(d2, 0.0) + 1e-12)
    mean_d = jnp.sum(jnp.where(wmask, sqrtd, zero), axis=1, keepdims=True) / K
    max_d = jnp.sqrt(jnp.maximum(v_cut, 0.0) + 1e-12)

    p0 = P[:, 0:1]
    p1 = P[:, 1:2]
    p2 = P[:, 2:3]
    colcat = jnp.concatenate(
        [csum(cx, cx), csum(cy, cy), csum(cz, cz),
         csum(cx, cy), csum(cx, cz), csum(cy, cz),
         p0, p1, p2, mean_d, max_d], axis=1)
    tr = jnp.transpose(colcat)
    A = {
        (0, 0): tr[0:1],
        (1, 1): tr[1:2],
        (2, 2): tr[2:3],
        (0, 1): tr[3:4],
        (0, 2): tr[4:5],
        (1, 2): tr[5:6],
    }
    p0r, p1r, p2r = tr[6:7], tr[7:8], tr[8:9]
    mean_dist = tr[9:10]
    max_dist = tr[10:11]

    one = jnp.ones((1, T), jnp.float32)
    zcol = jnp.zeros((1, T), jnp.float32)
    V = {(i, j): (one if i == j else zcol) for i in range(3) for j in range(3)}
    for _ in range(SWEEPS):
        for (p, q) in ((0, 2), (1, 2), (0, 1)):
            A, V = _jacobi_rotate(A, V, p, q)

    cols = [
        (A[(0, 0)], V[(0, 0)], V[(1, 0)], V[(2, 0)]),
        (A[(1, 1)], V[(0, 1)], V[(1, 1)], V[(2, 1)]),
        (A[(2, 2)], V[(0, 2)], V[(1, 2)], V[(2, 2)]),
    ]

    def cswap(a, b):
        swap = a[0] > b[0]
        lo = tuple(jnp.where(swap, yv, xv) for xv, yv in zip(a, b))
        hi = tuple(jnp.where(swap, xv, yv) for xv, yv in zip(a, b))
        return lo, hi

    cols[0], cols[1] = cswap(cols[0], cols[1])
    cols[1], cols[2] = cswap(cols[1], cols[2])
    cols[0], cols[1] = cswap(cols[0], cols[1])

    lam0, nx, ny, nz = cols[0]
    lam1 = cols[1][0]
    lam2 = cols[2][0]
    curv = lam0 / ((lam0 + lam1 + lam2) + 1e-9)

    xT = jnp.concatenate(
        [p0r, p1r, p2r, nx, ny, nz, curv, mean_dist, max_dist], axis=0)
    h1 = lax.dot_general(W1_ref[...], xT, (((1,), (0,)), ((), ())))
    h1 = jnp.maximum(h1 + b1_ref[...], 0.0)
    h2 = lax.dot_general(W2_ref[...], h1, (((1,), (0,)), ((), ())))
    h2 = jnp.maximum(h2 + b2_ref[...], 0.0)
    o = lax.dot_general(W3_ref[...], h2, (((1,), (0,)), ((), ())))
    out_ref[0] = o + b3_ref[...]


def kernel(point_cloud, vis_mask, W1, b1, W2, b2, W3, b3):
    B, N, _ = point_cloud.shape
    visible = jnp.where(vis_mask[..., None], point_cloud, jnp.zeros_like(point_cloud))
    ptsT = jnp.swapaxes(visible, 1, 2)
    sq = jnp.sum(visible * visible, axis=-1)
    sq_row = sq.reshape(B, 1, N)
    sq_col = sq.reshape(B, N, 1)
    C1, C2, C3 = W1.shape[0], W2.shape[0], W3.shape[0]
    b1c = b1.reshape(C1, 1)
    b2c = b2.reshape(C2, 1)
    b3c = b3.reshape(C3, 1)
    T = TILE
    grid = (B, N // T)
    out = pl.pallas_call(
        functools.partial(_fused_kernel, n=N),
        grid=grid,
        in_specs=[
            pl.BlockSpec((1, 3, N), lambda b, i: (b, 0, 0)),
            pl.BlockSpec((1, T, 3), lambda b, i: (b, i, 0)),
            pl.BlockSpec((1, 1, N), lambda b, i: (b, 0, 0)),
            pl.BlockSpec((1, T, 1), lambda b, i: (b, i, 0)),
            pl.BlockSpec((C1, 9), lambda b, i: (0, 0)),
            pl.BlockSpec((C1, 1), lambda b, i: (0, 0)),
            pl.BlockSpec((C2, C1), lambda b, i: (0, 0)),
            pl.BlockSpec((C2, 1), lambda b, i: (0, 0)),
            pl.BlockSpec((C3, C2), lambda b, i: (0, 0)),
            pl.BlockSpec((C3, 1), lambda b, i: (0, 0)),
        ],
        out_specs=pl.BlockSpec((1, C3, T), lambda b, i: (b, 0, i)),
        out_shape=jax.ShapeDtypeStruct((B, C3, N), jnp.float32),
    )(ptsT, visible, sq_row, sq_col, W1, b1c, W2, b2c, W3, b3c)
    return out
